# Initial kernel scaffold; baseline (speedup 1.0000x reference)
import jax
import jax.numpy as jnp
from jax.experimental import pallas as pl


def kernel(nodes, R, embed_s, embed_v):
    raise NotImplementedError("write your pallas kernel here")


# speedup vs baseline: 1.4764x; 1.4764x over previous
import functools

import jax
import jax.numpy as jnp
from jax import lax
from jax.experimental import pallas as pl
from jax.experimental.pallas import tpu as pltpu
from jax.experimental.pallas import tpu_sc as plsc

_NCOLS_S = 128
_NCOLS_V = 384
_VOCAB = 1000
_L = 16
_NW = 32
_MAX_NORM = 1.0
_EPS = 1e-7

_MESH = dict(core_axis_name="c", subcore_axis_name="s", num_cores=2,
             num_subcores=16)


def _worker_id():
    return lax.axis_index("s") * 2 + lax.axis_index("c")


def _rsqrt(x):
    i = lax.bitcast_convert_type(x, jnp.int32)
    i = jnp.int32(0x5F3759DF) - (i >> 1)
    y = lax.bitcast_convert_type(i, jnp.float32)
    for _ in range(3):
        y = y * (1.5 - 0.5 * x * y * y)
    return y


def _lane_sum(v):
    s = v[0]
    for i in range(1, _L):
        s = s + v[i]
    return s


def _max_norm_scale(nsq):
    norm = nsq * _rsqrt(nsq)
    rd = _rsqrt(norm + _EPS)
    return jnp.where(norm > _MAX_NORM, rd * rd * _MAX_NORM, 1.0)


_PREP_BC = 40
_PREP_NCHUNK = _VOCAB // _PREP_BC


def _prep_body(s_tab, v_tab, s_out, v_out, s_slab, v_slab, vo_slab, sem):
    w = _worker_id()
    iota3 = 3 * lax.iota(jnp.int32, _L)

    def do_chunk(t, carry):
        c = w + _NW * t

        @pl.when(c < _PREP_NCHUNK)
        def _():
            base = c * _PREP_BC
            pltpu.sync_copy(s_tab.at[pl.ds(base, _PREP_BC)], s_slab)
            pltpu.sync_copy(v_tab.at[pl.ds(base, _PREP_BC)], v_slab)

            def do_row(r, carry2):
                acc = jnp.zeros((_L,), jnp.float32)
                for g in range(_NCOLS_S // _L):
                    x = s_slab[r, pl.ds(g * _L, _L)]
                    acc = acc + x * x
                scale_s = _max_norm_scale(_lane_sum(acc))
                for g in range(_NCOLS_S // _L):
                    s_slab[r, pl.ds(g * _L, _L)] = (
                        s_slab[r, pl.ds(g * _L, _L)] * scale_s)
                acc = jnp.zeros((_L,), jnp.float32)
                for g in range(_NCOLS_V // _L):
                    x = v_slab[r, pl.ds(g * _L, _L)]
                    acc = acc + x * x
                scale_v = _max_norm_scale(_lane_sum(acc))
                row_idx = jnp.full((_L,), r, jnp.int32)
                for j in range(3):
                    for g in range(_NCOLS_S // _L):
                        src = plsc.load_gather(
                            v_slab, [row_idx, iota3 + (48 * g + j)])
                        vo_slab[r, pl.ds(j * _NCOLS_S + g * _L, _L)] = (
                            src * scale_v)
                return carry2

            lax.fori_loop(0, _PREP_BC, do_row, 0)
            pltpu.sync_copy(s_slab, s_out.at[pl.ds(base, _PREP_BC)])
            pltpu.sync_copy(vo_slab, v_out.at[pl.ds(base, _PREP_BC)])
        return carry

    lax.fori_loop(0, (_PREP_NCHUNK + _NW - 1) // _NW, do_chunk, 0)


@jax.jit
def _prep(embed_s, embed_v):
    fn = pl.kernel(
        _prep_body,
        out_type=[
            jax.ShapeDtypeStruct((_VOCAB, _NCOLS_S), jnp.float32),
            jax.ShapeDtypeStruct((_VOCAB, _NCOLS_V), jnp.float32),
        ],
        mesh=plsc.VectorSubcoreMesh(**_MESH),
        scratch_types=[
            pltpu.VMEM((_PREP_BC, _NCOLS_S), jnp.float32),
            pltpu.VMEM((_PREP_BC, _NCOLS_V), jnp.float32),
            pltpu.VMEM((_PREP_BC, _NCOLS_V), jnp.float32),
            pltpu.SemaphoreType.DMA,
        ],
        compiler_params=pltpu.CompilerParams(use_tc_tiling_on_sc=False, needs_layout_passes=False),
    )
    return fn(embed_s, embed_v)


_BC = 80


def _main_body(nodes, r16, s_tab, v_tab, out_s, out_rv,
               idx_v, r_buf, s_rows, v_rows, rv_buf, sem_g, sem_s):
    n = out_s.shape[0]
    nchunk = n // _BC
    w = _worker_id()
    iota3 = 3 * lax.iota(jnp.int32, _L)

    def do_chunk(t, carry):
        c = w + _NW * t

        @pl.when(c < nchunk)
        def _():
            base = c * _BC
            pltpu.sync_copy(nodes.at[pl.ds(base, _BC)], idx_v)
            g1 = pltpu.async_copy(s_tab.at[idx_v], s_rows, sem_g)
            g2 = pltpu.async_copy(v_tab.at[idx_v], v_rows, sem_g)
            pltpu.sync_copy(r16.at[pl.ds(base, _BC)], r_buf)
            g1.wait()
            g2.wait()
            cp_s = pltpu.async_copy(s_rows, out_s.at[pl.ds(base, _BC)], sem_s)

            def do_row(r, carry2):
                rvec = r_buf[r, pl.ds(0, _L)]
                rm = [[rvec[3 * i + j] for j in range(3)] for i in range(3)]
                row_idx = jnp.full((_L,), r, jnp.int32)
                for g in range(_NCOLS_S // _L):
                    a = v_rows[r, pl.ds(g * _L, _L)]
                    b = v_rows[r, pl.ds(_NCOLS_S + g * _L, _L)]
                    d = v_rows[r, pl.ds(2 * _NCOLS_S + g * _L, _L)]
                    for i in range(3):
                        o = rm[i][0] * a + rm[i][1] * b + rm[i][2] * d
                        plsc.store_scatter(
                            rv_buf, [row_idx, iota3 + (48 * g + i)], o)
                return carry2

            lax.fori_loop(0, _BC, do_row, 0)
            pltpu.sync_copy(rv_buf, out_rv.at[pl.ds(base, _BC)])
            cp_s.wait()
        return carry

    lax.fori_loop(0, (nchunk + _NW - 1) // _NW, do_chunk, 0)


@jax.jit
def _main(nodes, r16, s_tab, v_tab):
    n = nodes.shape[0]
    fn = pl.kernel(
        _main_body,
        out_type=[
            jax.ShapeDtypeStruct((n, _NCOLS_S), jnp.float32),
            jax.ShapeDtypeStruct((n, _NCOLS_V), jnp.float32),
        ],
        mesh=plsc.VectorSubcoreMesh(**_MESH),
        scratch_types=[
            pltpu.VMEM((_BC,), jnp.int32),
            pltpu.VMEM((_BC, _L), jnp.float32),
            pltpu.VMEM((_BC, _NCOLS_S), jnp.float32),
            pltpu.VMEM((_BC, _NCOLS_V), jnp.float32),
            pltpu.VMEM((_BC, _NCOLS_V), jnp.float32),
            pltpu.SemaphoreType.DMA,
            pltpu.SemaphoreType.DMA,
        ],
        compiler_params=pltpu.CompilerParams(use_tc_tiling_on_sc=False, needs_layout_passes=False),
    )
    return fn(nodes, r16, s_tab, v_tab)


def kernel(nodes, R, embed_s, embed_v):
    n = nodes.shape[0]
    nodes_i = nodes.astype(jnp.int32)
    r16 = jnp.pad(R.astype(jnp.float32).reshape(n, 9), ((0, 0), (0, 7)))
    s_tab, v_tab = _prep(embed_s, embed_v)
    s, rv = _main(nodes_i, r16, s_tab, v_tab)
    return s, rv.reshape(n, _NCOLS_S, 3)


# speedup vs baseline: 3.0835x; 2.0885x over previous
import functools

import jax
import jax.numpy as jnp
from jax import lax
from jax.experimental import pallas as pl
from jax.experimental.pallas import tpu as pltpu
from jax.experimental.pallas import tpu_sc as plsc

_NCOLS_S = 128
_NCOLS_V = 384
_VOCAB = 1000
_L = 16
_NW = 32
_MAX_NORM = 1.0
_EPS = 1e-7

_MESH = dict(core_axis_name="c", subcore_axis_name="s", num_cores=2,
             num_subcores=16)


def _worker_id():
    return lax.axis_index("s") * 2 + lax.axis_index("c")


def _rsqrt(x):
    i = lax.bitcast_convert_type(x, jnp.int32)
    i = jnp.int32(0x5F3759DF) - (i >> 1)
    y = lax.bitcast_convert_type(i, jnp.float32)
    for _ in range(3):
        y = y * (1.5 - 0.5 * x * y * y)
    return y


def _lane_sum(v):
    s = v[0]
    for i in range(1, _L):
        s = s + v[i]
    return s


def _max_norm_scale(nsq):
    norm = nsq * _rsqrt(nsq)
    rd = _rsqrt(norm + _EPS)
    return jnp.where(norm > _MAX_NORM, rd * rd * _MAX_NORM, 1.0)


_PREP_BC = 40
_PREP_NCHUNK = _VOCAB // _PREP_BC


def _prep_body(s_tab, v_tab, s_out, v_out, s_slab, v_slab, vo_slab, sem):
    w = _worker_id()
    iota3 = 3 * lax.iota(jnp.int32, _L)

    def do_chunk(t, carry):
        c = w + _NW * t

        @pl.when(c < _PREP_NCHUNK)
        def _():
            base = c * _PREP_BC
            pltpu.sync_copy(s_tab.at[pl.ds(base, _PREP_BC)], s_slab)
            pltpu.sync_copy(v_tab.at[pl.ds(base, _PREP_BC)], v_slab)

            def do_row(r, carry2):
                acc = jnp.zeros((_L,), jnp.float32)
                for g in range(_NCOLS_S // _L):
                    x = s_slab[r, pl.ds(g * _L, _L)]
                    acc = acc + x * x
                scale_s = _max_norm_scale(_lane_sum(acc))
                for g in range(_NCOLS_S // _L):
                    s_slab[r, pl.ds(g * _L, _L)] = (
                        s_slab[r, pl.ds(g * _L, _L)] * scale_s)
                acc = jnp.zeros((_L,), jnp.float32)
                for g in range(_NCOLS_V // _L):
                    x = v_slab[r, pl.ds(g * _L, _L)]
                    acc = acc + x * x
                scale_v = _max_norm_scale(_lane_sum(acc))
                row_idx = jnp.full((_L,), r, jnp.int32)
                for j in range(3):
                    for g in range(_NCOLS_S // _L):
                        src = plsc.load_gather(
                            v_slab, [row_idx, iota3 + (48 * g + j)])
                        vo_slab[r, pl.ds(j * _NCOLS_S + g * _L, _L)] = (
                            src * scale_v)
                return carry2

            lax.fori_loop(0, _PREP_BC, do_row, 0)
            pltpu.sync_copy(s_slab, s_out.at[pl.ds(base, _PREP_BC)])
            pltpu.sync_copy(vo_slab, v_out.at[pl.ds(base, _PREP_BC)])
        return carry

    lax.fori_loop(0, (_PREP_NCHUNK + _NW - 1) // _NW, do_chunk, 0)


@jax.jit
def _prep(embed_s, embed_v):
    fn = pl.kernel(
        _prep_body,
        out_type=[
            jax.ShapeDtypeStruct((_VOCAB, _NCOLS_S), jnp.float32),
            jax.ShapeDtypeStruct((_VOCAB, _NCOLS_V), jnp.float32),
        ],
        mesh=plsc.VectorSubcoreMesh(**_MESH),
        scratch_types=[
            pltpu.VMEM((_PREP_BC, _NCOLS_S), jnp.float32),
            pltpu.VMEM((_PREP_BC, _NCOLS_V), jnp.float32),
            pltpu.VMEM((_PREP_BC, _NCOLS_V), jnp.float32),
            pltpu.SemaphoreType.DMA,
        ],
        compiler_params=pltpu.CompilerParams(use_tc_tiling_on_sc=False, needs_layout_passes=False),
    )
    return fn(embed_s, embed_v)


_BC = 80


def _main_body(nodes, r16, s_tab, v_tab, out_s, out_rv,
               idx_v, r_buf, s_rows, v_rows, rv_buf, sem_g, sem_s):
    n = out_s.shape[0]
    nchunk = n // _BC
    w = _worker_id()

    def do_chunk(t, carry):
        c = w + _NW * t

        @pl.when(c < nchunk)
        def _():
            base = c * _BC
            pltpu.sync_copy(nodes.at[pl.ds(base, _BC)], idx_v)
            g1 = pltpu.async_copy(s_tab.at[idx_v], s_rows, sem_g)
            g2 = pltpu.async_copy(v_tab.at[idx_v], v_rows, sem_g)
            pltpu.sync_copy(r16.at[pl.ds(base, _BC)], r_buf)
            g1.wait()
            g2.wait()
            cp_s = pltpu.async_copy(s_rows, out_s.at[pl.ds(base, _BC)], sem_s)

            def do_row(r, carry2):
                rvec = r_buf[r, pl.ds(0, _L)]
                rm = [[rvec[3 * i + j] for j in range(3)] for i in range(3)]
                for g in range(_NCOLS_S // _L):
                    a = v_rows[r, pl.ds(g * _L, _L)]
                    b = v_rows[r, pl.ds(_NCOLS_S + g * _L, _L)]
                    d = v_rows[r, pl.ds(2 * _NCOLS_S + g * _L, _L)]
                    for i in range(3):
                        rv_buf[i, r, pl.ds(g * _L, _L)] = (
                            rm[i][0] * a + rm[i][1] * b + rm[i][2] * d)
                return carry2

            lax.fori_loop(0, _BC, do_row, 0)
            for i in range(3):
                pltpu.sync_copy(rv_buf.at[i],
                                out_rv.at[i, pl.ds(base, _BC)])
            cp_s.wait()
        return carry

    lax.fori_loop(0, (nchunk + _NW - 1) // _NW, do_chunk, 0)


@jax.jit
def _main(nodes, r16, s_tab, v_tab):
    n = nodes.shape[0]
    fn = pl.kernel(
        _main_body,
        out_type=[
            jax.ShapeDtypeStruct((n, _NCOLS_S), jnp.float32),
            jax.ShapeDtypeStruct((3, n, _NCOLS_S), jnp.float32),
        ],
        mesh=plsc.VectorSubcoreMesh(**_MESH),
        scratch_types=[
            pltpu.VMEM((_BC,), jnp.int32),
            pltpu.VMEM((_BC, _L), jnp.float32),
            pltpu.VMEM((_BC, _NCOLS_S), jnp.float32),
            pltpu.VMEM((_BC, _NCOLS_V), jnp.float32),
            pltpu.VMEM((3, _BC, _NCOLS_S), jnp.float32),
            pltpu.SemaphoreType.DMA,
            pltpu.SemaphoreType.DMA,
        ],
        compiler_params=pltpu.CompilerParams(use_tc_tiling_on_sc=False, needs_layout_passes=False),
    )
    return fn(nodes, r16, s_tab, v_tab)


def kernel(nodes, R, embed_s, embed_v):
    n = nodes.shape[0]
    nodes_i = nodes.astype(jnp.int32)
    r16 = jnp.pad(R.astype(jnp.float32).reshape(n, 9), ((0, 0), (0, 7)))
    s_tab, v_tab = _prep(embed_s, embed_v)
    s, rv_plan = _main(nodes_i, r16, s_tab, v_tab)
    return s, jnp.transpose(rv_plan, (1, 2, 0))


# speedup vs baseline: 3.6317x; 1.1778x over previous
import functools

import jax
import jax.numpy as jnp
from jax import lax
from jax.experimental import pallas as pl
from jax.experimental.pallas import tpu as pltpu
from jax.experimental.pallas import tpu_sc as plsc

_NCOLS_S = 128
_NCOLS_V = 384
_VOCAB = 1000
_L = 16
_NW = 32
_MAX_NORM = 1.0
_EPS = 1e-7

_MESH = dict(core_axis_name="c", subcore_axis_name="s", num_cores=2,
             num_subcores=16)


def _worker_id():
    return lax.axis_index("s") * 2 + lax.axis_index("c")


def _rsqrt(x):
    i = lax.bitcast_convert_type(x, jnp.int32)
    i = jnp.int32(0x5F3759DF) - (i >> 1)
    y = lax.bitcast_convert_type(i, jnp.float32)
    for _ in range(3):
        y = y * (1.5 - 0.5 * x * y * y)
    return y


def _lane_sum(v):
    s = v[0]
    for i in range(1, _L):
        s = s + v[i]
    return s


def _max_norm_scale(nsq):
    norm = nsq * _rsqrt(nsq)
    rd = _rsqrt(norm + _EPS)
    return jnp.where(norm > _MAX_NORM, rd * rd * _MAX_NORM, 1.0)


_PREP_BC = 40
_PREP_NCHUNK = _VOCAB // _PREP_BC


def _prep_body(s_tab, v_tab, s_out, v_out, s_slab, v_slab, vo_slab, sem):
    w = _worker_id()
    iota3 = 3 * lax.iota(jnp.int32, _L)

    def do_chunk(t, carry):
        c = w + _NW * t

        @pl.when(c < _PREP_NCHUNK)
        def _():
            base = c * _PREP_BC
            pltpu.sync_copy(s_tab.at[pl.ds(base, _PREP_BC)], s_slab)
            pltpu.sync_copy(v_tab.at[pl.ds(base, _PREP_BC)], v_slab)

            def do_row(r, carry2):
                acc = jnp.zeros((_L,), jnp.float32)
                for g in range(_NCOLS_S // _L):
                    x = s_slab[r, pl.ds(g * _L, _L)]
                    acc = acc + x * x
                scale_s = _max_norm_scale(_lane_sum(acc))
                for g in range(_NCOLS_S // _L):
                    s_slab[r, pl.ds(g * _L, _L)] = (
                        s_slab[r, pl.ds(g * _L, _L)] * scale_s)
                acc = jnp.zeros((_L,), jnp.float32)
                for g in range(_NCOLS_V // _L):
                    x = v_slab[r, pl.ds(g * _L, _L)]
                    acc = acc + x * x
                scale_v = _max_norm_scale(_lane_sum(acc))
                row_idx = jnp.full((_L,), r, jnp.int32)
                for j in range(3):
                    for g in range(_NCOLS_S // _L):
                        src = plsc.load_gather(
                            v_slab, [row_idx, iota3 + (48 * g + j)])
                        vo_slab[r, pl.ds(j * _NCOLS_S + g * _L, _L)] = (
                            src * scale_v)
                return carry2

            lax.fori_loop(0, _PREP_BC, do_row, 0)
            pltpu.sync_copy(s_slab, s_out.at[pl.ds(base, _PREP_BC)])
            pltpu.sync_copy(vo_slab, v_out.at[pl.ds(base, _PREP_BC)])
        return carry

    lax.fori_loop(0, (_PREP_NCHUNK + _NW - 1) // _NW, do_chunk, 0)


@jax.jit
def _prep(embed_s, embed_v):
    fn = pl.kernel(
        _prep_body,
        out_type=[
            jax.ShapeDtypeStruct((_VOCAB, _NCOLS_S), jnp.float32),
            jax.ShapeDtypeStruct((_VOCAB, _NCOLS_V), jnp.float32),
        ],
        mesh=plsc.VectorSubcoreMesh(**_MESH),
        scratch_types=[
            pltpu.VMEM((_PREP_BC, _NCOLS_S), jnp.float32),
            pltpu.VMEM((_PREP_BC, _NCOLS_V), jnp.float32),
            pltpu.VMEM((_PREP_BC, _NCOLS_V), jnp.float32),
            pltpu.SemaphoreType.DMA,
        ],
        compiler_params=pltpu.CompilerParams(use_tc_tiling_on_sc=False, needs_layout_passes=False),
    )
    return fn(embed_s, embed_v)


_BC = 40


def _main_body(nodes, r16, s_tab, v_tab, out_s, out_rv,
               idx_a, idx_b, r_a, r_b, s_a, s_b, v_a, v_b, rv_a, rv_b,
               sg_a, sg_b, so_a, so_b):
    n = out_s.shape[0]
    nchunk = n // _BC
    w = _worker_id()
    niter = (nchunk + _NW - 1) // _NW
    sets = [(idx_a, r_a, s_a, v_a, rv_a, sg_a, so_a),
            (idx_b, r_b, s_b, v_b, rv_b, sg_b, so_b)]

    def issue_inputs(c, st):
        idx, rb, sr, vr, rv, sg, so = st
        pltpu.sync_copy(nodes.at[pl.ds(c * _BC, _BC)], idx)
        pltpu.async_copy(s_tab.at[idx], sr, sg)
        pltpu.async_copy(v_tab.at[idx], vr, sg)
        pltpu.async_copy(r16.at[pl.ds(c * _BC, _BC)], rb, sg)

    def drain_inputs(st):
        idx, rb, sr, vr, rv, sg, so = st
        pltpu.make_async_copy(s_tab.at[idx], sr, sg).wait()
        pltpu.make_async_copy(v_tab.at[idx], vr, sg).wait()
        pltpu.make_async_copy(r16.at[pl.ds(0, _BC)], rb, sg).wait()

    def drain_outputs(st):
        idx, rb, sr, vr, rv, sg, so = st
        pltpu.make_async_copy(sr, out_s.at[pl.ds(0, _BC)], so).wait()
        for i in range(3):
            pltpu.make_async_copy(
                rv.at[i], out_rv.at[i, pl.ds(0, _BC)], so).wait()

    def compute_and_out(c, st):
        idx, rb, sr, vr, rv, sg, so = st
        base = c * _BC

        def do_row(r, carry2):
            rvec = rb[r, pl.ds(0, _L)]
            rm = [[rvec[3 * i + j] for j in range(3)] for i in range(3)]
            for g in range(_NCOLS_S // _L):
                a = vr[r, pl.ds(g * _L, _L)]
                b = vr[r, pl.ds(_NCOLS_S + g * _L, _L)]
                d = vr[r, pl.ds(2 * _NCOLS_S + g * _L, _L)]
                for i in range(3):
                    rv[i, r, pl.ds(g * _L, _L)] = (
                        rm[i][0] * a + rm[i][1] * b + rm[i][2] * d)
            return carry2

        lax.fori_loop(0, _BC, do_row, 0)
        pltpu.async_copy(sr, out_s.at[pl.ds(base, _BC)], so)
        for i in range(3):
            pltpu.async_copy(rv.at[i], out_rv.at[i, pl.ds(base, _BC)], so)

    issue_inputs(w, sets[0])

    def outer(to, carry):
        for par in range(2):
            t = 2 * to + par
            cur = sets[par]
            nxt = sets[1 - par]
            c_now = w + _NW * t
            c_next = c_now + _NW

            @pl.when(c_now < nchunk)
            def _():
                drain_inputs(cur)

            @pl.when(jnp.logical_and(t >= 1, c_next < nchunk))
            def _():
                drain_outputs(nxt)

            @pl.when(c_next < nchunk)
            def _():
                issue_inputs(c_next, nxt)

            @pl.when(c_now < nchunk)
            def _():
                compute_and_out(c_now, cur)
        return carry

    lax.fori_loop(0, (niter + 2) // 2, outer, 0)
    drain_outputs(sets[0])
    drain_outputs(sets[1])


@jax.jit
def _main(nodes, r16, s_tab, v_tab):
    n = nodes.shape[0]
    dbl = lambda shape, dt: [pltpu.VMEM(shape, dt), pltpu.VMEM(shape, dt)]
    fn = pl.kernel(
        _main_body,
        out_type=[
            jax.ShapeDtypeStruct((n, _NCOLS_S), jnp.float32),
            jax.ShapeDtypeStruct((3, n, _NCOLS_S), jnp.float32),
        ],
        mesh=plsc.VectorSubcoreMesh(**_MESH),
        scratch_types=(
            dbl((_BC,), jnp.int32)
            + dbl((_BC, _L), jnp.float32)
            + dbl((_BC, _NCOLS_S), jnp.float32)
            + dbl((_BC, _NCOLS_V), jnp.float32)
            + dbl((3, _BC, _NCOLS_S), jnp.float32)
            + [pltpu.SemaphoreType.DMA] * 4
        ),
        compiler_params=pltpu.CompilerParams(use_tc_tiling_on_sc=False, needs_layout_passes=False),
    )
    return fn(nodes, r16, s_tab, v_tab)


def kernel(nodes, R, embed_s, embed_v):
    n = nodes.shape[0]
    nodes_i = nodes.astype(jnp.int32)
    r16 = jnp.pad(R.astype(jnp.float32).reshape(n, 9), ((0, 0), (0, 7)))
    s_tab, v_tab = _prep(embed_s, embed_v)
    s, rv_plan = _main(nodes_i, r16, s_tab, v_tab)
    return s, jnp.transpose(rv_plan, (1, 2, 0))


# speedup vs baseline: 3.8901x; 1.0711x over previous
import functools

import jax
import jax.numpy as jnp
from jax import lax
from jax.experimental import pallas as pl
from jax.experimental.pallas import tpu as pltpu
from jax.experimental.pallas import tpu_sc as plsc

_NCOLS_S = 128
_NCOLS_V = 384
_VOCAB = 1000
_L = 16
_NW = 32
_MAX_NORM = 1.0
_EPS = 1e-7

_MESH = dict(core_axis_name="c", subcore_axis_name="s", num_cores=2,
             num_subcores=16)


def _worker_id():
    return lax.axis_index("s") * 2 + lax.axis_index("c")


def _rsqrt(x):
    i = lax.bitcast_convert_type(x, jnp.int32)
    i = jnp.int32(0x5F3759DF) - (i >> 1)
    y = lax.bitcast_convert_type(i, jnp.float32)
    for _ in range(3):
        y = y * (1.5 - 0.5 * x * y * y)
    return y


def _lane_sum(v):
    s = v[0]
    for i in range(1, _L):
        s = s + v[i]
    return s


def _max_norm_scale(nsq):
    norm = nsq * _rsqrt(nsq)
    rd = _rsqrt(norm + _EPS)
    return jnp.where(norm > _MAX_NORM, rd * rd * _MAX_NORM, 1.0)


_PREP_BC = 40
_PREP_NCHUNK = _VOCAB // _PREP_BC


def _prep_body(s_tab, v_tab, s_out, v_out, s_slab, v_slab, vo_slab, sem):
    w = _worker_id()
    iota3 = 3 * lax.iota(jnp.int32, _L)

    def do_chunk(t, carry):
        c = w + _NW * t

        @pl.when(c < _PREP_NCHUNK)
        def _():
            base = c * _PREP_BC
            pltpu.sync_copy(s_tab.at[pl.ds(base, _PREP_BC)], s_slab)
            pltpu.sync_copy(v_tab.at[pl.ds(base, _PREP_BC)], v_slab)

            def do_row(r, carry2):
                acc = jnp.zeros((_L,), jnp.float32)
                for g in range(_NCOLS_S // _L):
                    x = s_slab[r, pl.ds(g * _L, _L)]
                    acc = acc + x * x
                scale_s = _max_norm_scale(_lane_sum(acc))
                for g in range(_NCOLS_S // _L):
                    s_slab[r, pl.ds(g * _L, _L)] = (
                        s_slab[r, pl.ds(g * _L, _L)] * scale_s)
                acc = jnp.zeros((_L,), jnp.float32)
                for g in range(_NCOLS_V // _L):
                    x = v_slab[r, pl.ds(g * _L, _L)]
                    acc = acc + x * x
                scale_v = _max_norm_scale(_lane_sum(acc))
                row_idx = jnp.full((_L,), r, jnp.int32)
                for j in range(3):
                    for g in range(_NCOLS_S // _L):
                        src = plsc.load_gather(
                            v_slab, [row_idx, iota3 + (48 * g + j)])
                        vo_slab[r, pl.ds(j * _NCOLS_S + g * _L, _L)] = (
                            src * scale_v)
                return carry2

            lax.fori_loop(0, _PREP_BC, do_row, 0)
            pltpu.sync_copy(s_slab, s_out.at[pl.ds(base, _PREP_BC)])
            pltpu.sync_copy(vo_slab, v_out.at[pl.ds(base, _PREP_BC)])
        return carry

    lax.fori_loop(0, (_PREP_NCHUNK + _NW - 1) // _NW, do_chunk, 0)


@jax.jit
def _prep(embed_s, embed_v):
    fn = pl.kernel(
        _prep_body,
        out_type=[
            jax.ShapeDtypeStruct((_VOCAB, _NCOLS_S), jnp.float32),
            jax.ShapeDtypeStruct((_VOCAB, _NCOLS_V), jnp.float32),
        ],
        mesh=plsc.VectorSubcoreMesh(**_MESH),
        scratch_types=[
            pltpu.VMEM((_PREP_BC, _NCOLS_S), jnp.float32),
            pltpu.VMEM((_PREP_BC, _NCOLS_V), jnp.float32),
            pltpu.VMEM((_PREP_BC, _NCOLS_V), jnp.float32),
            pltpu.SemaphoreType.DMA,
        ],
        compiler_params=pltpu.CompilerParams(use_tc_tiling_on_sc=False, needs_layout_passes=False),
    )
    return fn(embed_s, embed_v)


_BC = 40


def _main_body(nodes, r16, s_tab, v_tab, out_s, out_rv,
               idx_a, idx_b, r_a, r_b, s_a, s_b, v_a, v_b, rv_a, rv_b,
               sh_nodes, bounce_i, sg_a, sg_b, so_a, so_b):
    n = out_s.shape[0]
    nchunk = n // _BC
    w = _worker_id()
    niter = (nchunk + _NW - 1) // _NW
    sets = [(idx_a, r_a, s_a, v_a, rv_a, sg_a, so_a),
            (idx_b, r_b, s_b, v_b, rv_b, sg_b, so_b)]

    sid = lax.axis_index("s")

    @pl.when(sid < 10)
    def _():
        no = sid * (n // 10)
        pltpu.sync_copy(nodes.at[pl.ds(no, n // 10)], bounce_i)
        pltpu.sync_copy(bounce_i, sh_nodes.at[pl.ds(no, n // 10)])

    plsc.subcore_barrier()

    def issue_inputs(c, st):
        idx, rb, sr, vr, rv, sg, so = st
        pltpu.sync_copy(sh_nodes.at[pl.ds(c * _BC, _BC)], idx)
        pltpu.async_copy(s_tab.at[idx], sr, sg)
        pltpu.async_copy(v_tab.at[idx], vr, sg)
        pltpu.async_copy(r16.at[pl.ds(c * _BC, _BC)], rb, sg)

    def drain_inputs(st):
        idx, rb, sr, vr, rv, sg, so = st
        pltpu.make_async_copy(s_tab.at[idx], sr, sg).wait()
        pltpu.make_async_copy(v_tab.at[idx], vr, sg).wait()
        pltpu.make_async_copy(r16.at[pl.ds(0, _BC)], rb, sg).wait()

    def drain_outputs(st):
        idx, rb, sr, vr, rv, sg, so = st
        pltpu.make_async_copy(sr, out_s.at[pl.ds(0, _BC)], so).wait()
        for i in range(3):
            pltpu.make_async_copy(
                rv.at[i], out_rv.at[i, pl.ds(0, _BC)], so).wait()

    def compute_and_out(c, st):
        idx, rb, sr, vr, rv, sg, so = st
        base = c * _BC

        def do_row(r, carry2):
            rvec = rb[r, pl.ds(0, _L)]
            rm = [[rvec[3 * i + j] for j in range(3)] for i in range(3)]
            for g in range(_NCOLS_S // _L):
                a = vr[r, pl.ds(g * _L, _L)]
                b = vr[r, pl.ds(_NCOLS_S + g * _L, _L)]
                d = vr[r, pl.ds(2 * _NCOLS_S + g * _L, _L)]
                for i in range(3):
                    rv[i, r, pl.ds(g * _L, _L)] = (
                        rm[i][0] * a + rm[i][1] * b + rm[i][2] * d)
            return carry2

        lax.fori_loop(0, _BC, do_row, 0)
        pltpu.async_copy(sr, out_s.at[pl.ds(base, _BC)], so)
        for i in range(3):
            pltpu.async_copy(rv.at[i], out_rv.at[i, pl.ds(base, _BC)], so)

    issue_inputs(w, sets[0])

    def outer(to, carry):
        for par in range(2):
            t = 2 * to + par
            cur = sets[par]
            nxt = sets[1 - par]
            c_now = w + _NW * t
            c_next = c_now + _NW

            @pl.when(c_now < nchunk)
            def _():
                drain_inputs(cur)

            @pl.when(jnp.logical_and(t >= 1, c_next < nchunk))
            def _():
                drain_outputs(nxt)

            @pl.when(c_next < nchunk)
            def _():
                issue_inputs(c_next, nxt)

            @pl.when(c_now < nchunk)
            def _():
                compute_and_out(c_now, cur)
        return carry

    lax.fori_loop(0, (niter + 2) // 2, outer, 0)
    drain_outputs(sets[0])
    drain_outputs(sets[1])


@jax.jit
def _main(nodes, r16, s_tab, v_tab):
    n = nodes.shape[0]
    dbl = lambda shape, dt: [pltpu.VMEM(shape, dt), pltpu.VMEM(shape, dt)]
    fn = pl.kernel(
        _main_body,
        out_type=[
            jax.ShapeDtypeStruct((n, _NCOLS_S), jnp.float32),
            jax.ShapeDtypeStruct((3, n, _NCOLS_S), jnp.float32),
        ],
        mesh=plsc.VectorSubcoreMesh(**_MESH),
        scratch_types=(
            dbl((_BC,), jnp.int32)
            + dbl((_BC, _L), jnp.float32)
            + dbl((_BC, _NCOLS_S), jnp.float32)
            + dbl((_BC, _NCOLS_V), jnp.float32)
            + dbl((3, _BC, _NCOLS_S), jnp.float32)
            + [pltpu.VMEM_SHARED((n,), jnp.int32),
               pltpu.VMEM((n // 10,), jnp.int32)]
            + [pltpu.SemaphoreType.DMA] * 4
        ),
        compiler_params=pltpu.CompilerParams(use_tc_tiling_on_sc=False, needs_layout_passes=False),
    )
    return fn(nodes, r16, s_tab, v_tab)


def kernel(nodes, R, embed_s, embed_v):
    n = nodes.shape[0]
    nodes_i = nodes.astype(jnp.int32)
    r16 = jnp.pad(R.astype(jnp.float32).reshape(n, 9), ((0, 0), (0, 7)))
    s_tab, v_tab = _prep(embed_s, embed_v)
    s, rv_plan = _main(nodes_i, r16, s_tab, v_tab)
    return s, jnp.transpose(rv_plan, (1, 2, 0))


# speedup vs baseline: 4.1689x; 1.0717x over previous
import functools

import jax
import jax.numpy as jnp
from jax import lax
from jax.experimental import pallas as pl
from jax.experimental.pallas import tpu as pltpu
from jax.experimental.pallas import tpu_sc as plsc

_NCOLS_S = 128
_NCOLS_V = 384
_VOCAB = 1000
_L = 16
_NW = 32
_MAX_NORM = 1.0
_EPS = 1e-7

_MESH = dict(core_axis_name="c", subcore_axis_name="s", num_cores=2,
             num_subcores=16)


def _worker_id():
    return lax.axis_index("s") * 2 + lax.axis_index("c")


def _rsqrt(x):
    i = lax.bitcast_convert_type(x, jnp.int32)
    i = jnp.int32(0x5F3759DF) - (i >> 1)
    y = lax.bitcast_convert_type(i, jnp.float32)
    for _ in range(3):
        y = y * (1.5 - 0.5 * x * y * y)
    return y


def _lane_sum(v):
    s = v[0]
    for i in range(1, _L):
        s = s + v[i]
    return s


def _max_norm_scale(nsq):
    norm = nsq * _rsqrt(nsq)
    rd = _rsqrt(norm + _EPS)
    return jnp.where(norm > _MAX_NORM, rd * rd * _MAX_NORM, 1.0)


_PREP_BC = 40
_PREP_NCHUNK = _VOCAB // _PREP_BC


def _prep_body(s_tab, v_tab, s_out, v_out, s_slab, v_slab, vo_slab, sem):
    w = _worker_id()
    iota3 = 3 * lax.iota(jnp.int32, _L)

    def do_chunk(t, carry):
        c = w + _NW * t

        @pl.when(c < _PREP_NCHUNK)
        def _():
            base = c * _PREP_BC
            pltpu.sync_copy(s_tab.at[pl.ds(base, _PREP_BC)], s_slab)
            pltpu.sync_copy(v_tab.at[pl.ds(base, _PREP_BC)], v_slab)

            def do_row(r, carry2):
                acc = jnp.zeros((_L,), jnp.float32)
                for g in range(_NCOLS_S // _L):
                    x = s_slab[r, pl.ds(g * _L, _L)]
                    acc = acc + x * x
                scale_s = _max_norm_scale(_lane_sum(acc))
                for g in range(_NCOLS_S // _L):
                    s_slab[r, pl.ds(g * _L, _L)] = (
                        s_slab[r, pl.ds(g * _L, _L)] * scale_s)
                acc = jnp.zeros((_L,), jnp.float32)
                for g in range(_NCOLS_V // _L):
                    x = v_slab[r, pl.ds(g * _L, _L)]
                    acc = acc + x * x
                scale_v = _max_norm_scale(_lane_sum(acc))
                row_idx = jnp.full((_L,), r, jnp.int32)
                for j in range(3):
                    for g in range(_NCOLS_S // _L):
                        src = plsc.load_gather(
                            v_slab, [row_idx, iota3 + (48 * g + j)])
                        vo_slab[r, pl.ds(j * _NCOLS_S + g * _L, _L)] = (
                            src * scale_v)
                return carry2

            lax.fori_loop(0, _PREP_BC, do_row, 0)
            pltpu.sync_copy(s_slab, s_out.at[pl.ds(base, _PREP_BC)])
            pltpu.sync_copy(vo_slab, v_out.at[pl.ds(base, _PREP_BC)])
        return carry

    lax.fori_loop(0, (_PREP_NCHUNK + _NW - 1) // _NW, do_chunk, 0)


@jax.jit
def _prep(embed_s, embed_v):
    fn = pl.kernel(
        _prep_body,
        out_type=[
            jax.ShapeDtypeStruct((_VOCAB, _NCOLS_S), jnp.float32),
            jax.ShapeDtypeStruct((_VOCAB, _NCOLS_V), jnp.float32),
        ],
        mesh=plsc.VectorSubcoreMesh(**_MESH),
        scratch_types=[
            pltpu.VMEM((_PREP_BC, _NCOLS_S), jnp.float32),
            pltpu.VMEM((_PREP_BC, _NCOLS_V), jnp.float32),
            pltpu.VMEM((_PREP_BC, _NCOLS_V), jnp.float32),
            pltpu.SemaphoreType.DMA,
        ],
        compiler_params=pltpu.CompilerParams(use_tc_tiling_on_sc=False, needs_layout_passes=False),
    )
    return fn(embed_s, embed_v)


_BC = 40


def _main_body(nodes, r16, s_tab, v_tab, out_s, out_rv,
               idx_a, idx_b, r_a, r_b, s_a, s_b, v_a, v_b, rv_a, rv_b,
               sh_nodes, bounce_i, sg_a, sg_b, ss_a, ss_b, sr_a, sr_b):
    n = out_s.shape[0]
    nchunk = n // _BC
    w = _worker_id()
    niter = (nchunk + _NW - 1) // _NW
    sets = [(idx_a, r_a, s_a, v_a, rv_a, sg_a, ss_a, sr_a),
            (idx_b, r_b, s_b, v_b, rv_b, sg_b, ss_b, sr_b)]

    sid = lax.axis_index("s")

    @pl.when(sid < 10)
    def _():
        no = sid * (n // 10)
        pltpu.sync_copy(nodes.at[pl.ds(no, n // 10)], bounce_i)
        pltpu.sync_copy(bounce_i, sh_nodes.at[pl.ds(no, n // 10)])

    plsc.subcore_barrier()

    def issue_inputs(c, st):
        idx, rb, sr, vr, rv, sg, ss, so = st
        pltpu.sync_copy(sh_nodes.at[pl.ds(c * _BC, _BC)], idx)
        pltpu.async_copy(s_tab.at[idx], sr, sg)
        pltpu.async_copy(v_tab.at[idx], vr, sg)
        pltpu.async_copy(r16.at[pl.ds(c * _BC, _BC)], rb, sg)

    def drain_inputs(st):
        idx, rb, sr, vr, rv, sg, ss, so = st
        pltpu.make_async_copy(s_tab.at[idx], sr, sg).wait()
        pltpu.make_async_copy(v_tab.at[idx], vr, sg).wait()
        pltpu.make_async_copy(r16.at[pl.ds(0, _BC)], rb, sg).wait()

    def drain_s_out(st):
        idx, rb, sr, vr, rv, sg, ss, so = st
        pltpu.make_async_copy(sr, out_s.at[pl.ds(0, _BC)], ss).wait()

    def drain_rv_out(st):
        idx, rb, sr, vr, rv, sg, ss, so = st
        for i in range(3):
            pltpu.make_async_copy(
                rv.at[i], out_rv.at[i, pl.ds(0, _BC)], so).wait()

    def compute_and_out(c, st):
        idx, rb, sr, vr, rv, sg, ss, so = st
        base = c * _BC

        def do_row(r, carry2):
            rvec = rb[r, pl.ds(0, _L)]
            rm = [[rvec[3 * i + j] for j in range(3)] for i in range(3)]
            for g in range(_NCOLS_S // _L):
                a = vr[r, pl.ds(g * _L, _L)]
                b = vr[r, pl.ds(_NCOLS_S + g * _L, _L)]
                d = vr[r, pl.ds(2 * _NCOLS_S + g * _L, _L)]
                for i in range(3):
                    rv[i, r, pl.ds(g * _L, _L)] = (
                        rm[i][0] * a + rm[i][1] * b + rm[i][2] * d)
            return carry2

        lax.fori_loop(0, _BC, do_row, 0)
        pltpu.async_copy(sr, out_s.at[pl.ds(base, _BC)], ss)
        for i in range(3):
            pltpu.async_copy(rv.at[i], out_rv.at[i, pl.ds(base, _BC)], so)

    issue_inputs(w, sets[0])

    def outer(to, carry):
        for par in range(2):
            t = 2 * to + par
            cur = sets[par]
            nxt = sets[1 - par]
            c_now = w + _NW * t
            c_next = c_now + _NW

            @pl.when(c_now < nchunk)
            def _():
                drain_inputs(cur)

            @pl.when(jnp.logical_and(t >= 1, c_now - _NW < nchunk))
            def _():
                drain_s_out(nxt)

            @pl.when(c_next < nchunk)
            def _():
                issue_inputs(c_next, nxt)

            @pl.when(jnp.logical_and(t >= 2, c_now - 2 * _NW < nchunk))
            def _():
                drain_rv_out(cur)

            @pl.when(c_now < nchunk)
            def _():
                compute_and_out(c_now, cur)
        return carry

    lax.fori_loop(0, (niter + 2) // 2, outer, 0)

    @pl.when(w + _NW * (2 * ((niter + 2) // 2) - 2) < nchunk)
    def _():
        drain_rv_out(sets[0])


@jax.jit
def _main(nodes, r16, s_tab, v_tab):
    n = nodes.shape[0]
    dbl = lambda shape, dt: [pltpu.VMEM(shape, dt), pltpu.VMEM(shape, dt)]
    fn = pl.kernel(
        _main_body,
        out_type=[
            jax.ShapeDtypeStruct((n, _NCOLS_S), jnp.float32),
            jax.ShapeDtypeStruct((3, n, _NCOLS_S), jnp.float32),
        ],
        mesh=plsc.VectorSubcoreMesh(**_MESH),
        scratch_types=(
            dbl((_BC,), jnp.int32)
            + dbl((_BC, _L), jnp.float32)
            + dbl((_BC, _NCOLS_S), jnp.float32)
            + dbl((_BC, _NCOLS_V), jnp.float32)
            + dbl((3, _BC, _NCOLS_S), jnp.float32)
            + [pltpu.VMEM_SHARED((n,), jnp.int32),
               pltpu.VMEM((n // 10,), jnp.int32)]
            + [pltpu.SemaphoreType.DMA] * 6
        ),
        compiler_params=pltpu.CompilerParams(use_tc_tiling_on_sc=False, needs_layout_passes=False),
    )
    return fn(nodes, r16, s_tab, v_tab)


def kernel(nodes, R, embed_s, embed_v):
    n = nodes.shape[0]
    nodes_i = nodes.astype(jnp.int32)
    r16 = jnp.pad(R.astype(jnp.float32).reshape(n, 9), ((0, 0), (0, 7)))
    s_tab, v_tab = _prep(embed_s, embed_v)
    s, rv_plan = _main(nodes_i, r16, s_tab, v_tab)
    return s, jnp.transpose(rv_plan, (1, 2, 0))


# speedup vs baseline: 5.4628x; 1.3104x over previous
import functools

import jax
import jax.numpy as jnp
from jax import lax
from jax.experimental import pallas as pl
from jax.experimental.pallas import tpu as pltpu
from jax.experimental.pallas import tpu_sc as plsc

_NCOLS_S = 128
_NCOLS_V = 384
_VOCAB = 1000
_L = 16
_NW = 32
_MAX_NORM = 1.0
_EPS = 1e-7

_MESH = dict(core_axis_name="c", subcore_axis_name="s", num_cores=2,
             num_subcores=16)


def _worker_id():
    return lax.axis_index("s") * 2 + lax.axis_index("c")


def _rsqrt(x):
    i = lax.bitcast_convert_type(x, jnp.int32)
    i = jnp.int32(0x5F3759DF) - (i >> 1)
    y = lax.bitcast_convert_type(i, jnp.float32)
    for _ in range(3):
        y = y * (1.5 - 0.5 * x * y * y)
    return y


def _lane_sum(v):
    s = v[0]
    for i in range(1, _L):
        s = s + v[i]
    return s


def _max_norm_scale(nsq):
    norm = nsq * _rsqrt(nsq)
    rd = _rsqrt(norm + _EPS)
    return jnp.where(norm > _MAX_NORM, rd * rd * _MAX_NORM, 1.0)


_PREP_BC = 40
_PREP_NCHUNK = _VOCAB // _PREP_BC


def _prep_body(s_tab, v_tab, s_out, v_out, s_slab, v_slab, vo_slab, sem):
    w = _worker_id()
    iota3 = 3 * lax.iota(jnp.int32, _L)

    def do_chunk(t, carry):
        c = w + _NW * t

        @pl.when(c < _PREP_NCHUNK)
        def _():
            base = c * _PREP_BC
            pltpu.sync_copy(s_tab.at[pl.ds(base, _PREP_BC)], s_slab)
            pltpu.sync_copy(v_tab.at[pl.ds(base, _PREP_BC)], v_slab)

            def do_row(r, carry2):
                acc = jnp.zeros((_L,), jnp.float32)
                for g in range(_NCOLS_S // _L):
                    x = s_slab[r, pl.ds(g * _L, _L)]
                    acc = acc + x * x
                scale_s = _max_norm_scale(_lane_sum(acc))
                for g in range(_NCOLS_S // _L):
                    s_slab[r, pl.ds(g * _L, _L)] = (
                        s_slab[r, pl.ds(g * _L, _L)] * scale_s)
                acc = jnp.zeros((_L,), jnp.float32)
                for g in range(_NCOLS_V // _L):
                    x = v_slab[r, pl.ds(g * _L, _L)]
                    acc = acc + x * x
                scale_v = _max_norm_scale(_lane_sum(acc))
                row_idx = jnp.full((_L,), r, jnp.int32)
                for j in range(3):
                    for g in range(_NCOLS_S // _L):
                        src = plsc.load_gather(
                            v_slab, [row_idx, iota3 + (48 * g + j)])
                        vo_slab[r, pl.ds(j * _NCOLS_S + g * _L, _L)] = (
                            src * scale_v)
                return carry2

            lax.fori_loop(0, _PREP_BC, do_row, 0)
            pltpu.sync_copy(s_slab, s_out.at[pl.ds(base, _PREP_BC)])
            pltpu.sync_copy(vo_slab, v_out.at[pl.ds(base, _PREP_BC)])
        return carry

    lax.fori_loop(0, (_PREP_NCHUNK + _NW - 1) // _NW, do_chunk, 0)


@jax.jit
def _prep(embed_s, embed_v):
    fn = pl.kernel(
        _prep_body,
        out_type=[
            jax.ShapeDtypeStruct((_VOCAB, _NCOLS_S), jnp.float32),
            jax.ShapeDtypeStruct((_VOCAB, _NCOLS_V), jnp.float32),
        ],
        mesh=plsc.VectorSubcoreMesh(**_MESH),
        scratch_types=[
            pltpu.VMEM((_PREP_BC, _NCOLS_S), jnp.float32),
            pltpu.VMEM((_PREP_BC, _NCOLS_V), jnp.float32),
            pltpu.VMEM((_PREP_BC, _NCOLS_V), jnp.float32),
            pltpu.SemaphoreType.DMA,
        ],
        compiler_params=pltpu.CompilerParams(use_tc_tiling_on_sc=False, needs_layout_passes=False),
    )
    return fn(embed_s, embed_v)


_BC = 40


def _main_body(nodes, r16, s_tab, v_tab, out_s, out_rv,
               idx_a, idx_b, r_a, r_b, s_a, s_b, v_a, v_b, rv_a, rv_b,
               sh_nodes, bounce_i, sg_a, sg_b, ss_a, ss_b, sr_a, sr_b):
    n = out_s.shape[0]
    nchunk = n // _BC
    w = _worker_id()
    niter = (nchunk + _NW - 1) // _NW
    sets = [(idx_a, r_a, s_a, v_a, rv_a, sg_a, ss_a, sr_a),
            (idx_b, r_b, s_b, v_b, rv_b, sg_b, ss_b, sr_b)]

    sid = lax.axis_index("s")

    @pl.when(sid < 10)
    def _():
        no = sid * (n // 10)
        pltpu.sync_copy(nodes.at[pl.ds(no, n // 10)], bounce_i)
        pltpu.sync_copy(bounce_i, sh_nodes.at[pl.ds(no, n // 10)])

    plsc.subcore_barrier()

    def issue_inputs(c, st):
        idx, rb, sr, vr, rv, sg, ss, so = st
        pltpu.sync_copy(sh_nodes.at[pl.ds(c * _BC, _BC)], idx)
        pltpu.async_copy(s_tab.at[idx], sr, sg)
        pltpu.async_copy(v_tab.at[idx], vr, sg)
        pltpu.async_copy(r16.at[pl.ds(c * _BC, _BC)], rb, sg)

    def drain_inputs(st):
        idx, rb, sr, vr, rv, sg, ss, so = st
        pltpu.make_async_copy(s_tab.at[idx], sr, sg).wait()
        pltpu.make_async_copy(v_tab.at[idx], vr, sg).wait()
        pltpu.make_async_copy(r16.at[pl.ds(0, _BC)], rb, sg).wait()

    def drain_s_out(st):
        idx, rb, sr, vr, rv, sg, ss, so = st
        pltpu.make_async_copy(sr, out_s.at[pl.ds(0, _BC)], ss).wait()

    def drain_rv_out(st):
        idx, rb, sr, vr, rv, sg, ss, so = st
        for i in range(3):
            pltpu.make_async_copy(
                rv.at[i], out_rv.at[i, pl.ds(0, _BC)], so).wait()

    def compute_and_out(c, st):
        idx, rb, sr, vr, rv, sg, ss, so = st
        base = c * _BC

        @plsc.parallel_loop(0, _BC, 1, unroll=2)
        def do_row(r):
            rvec = rb[r, pl.ds(0, _L)]
            rm = [[rvec[3 * i + j] for j in range(3)] for i in range(3)]
            for g in range(_NCOLS_S // _L):
                a = vr[r, pl.ds(g * _L, _L)]
                b = vr[r, pl.ds(_NCOLS_S + g * _L, _L)]
                d = vr[r, pl.ds(2 * _NCOLS_S + g * _L, _L)]
                for i in range(3):
                    rv[i, r, pl.ds(g * _L, _L)] = (
                        rm[i][0] * a + rm[i][1] * b + rm[i][2] * d)
        pltpu.async_copy(sr, out_s.at[pl.ds(base, _BC)], ss)
        for i in range(3):
            pltpu.async_copy(rv.at[i], out_rv.at[i, pl.ds(base, _BC)], so)

    issue_inputs(w, sets[0])

    def outer(to, carry):
        for par in range(2):
            t = 2 * to + par
            cur = sets[par]
            nxt = sets[1 - par]
            c_now = w + _NW * t
            c_next = c_now + _NW

            @pl.when(c_now < nchunk)
            def _():
                drain_inputs(cur)

            @pl.when(jnp.logical_and(t >= 1, c_now - _NW < nchunk))
            def _():
                drain_s_out(nxt)

            @pl.when(c_next < nchunk)
            def _():
                issue_inputs(c_next, nxt)

            @pl.when(jnp.logical_and(t >= 2, c_now - 2 * _NW < nchunk))
            def _():
                drain_rv_out(cur)

            @pl.when(c_now < nchunk)
            def _():
                compute_and_out(c_now, cur)
        return carry

    lax.fori_loop(0, (niter + 2) // 2, outer, 0)

    @pl.when(w + _NW * (2 * ((niter + 2) // 2) - 2) < nchunk)
    def _():
        drain_rv_out(sets[0])


@jax.jit
def _main(nodes, r16, s_tab, v_tab):
    n = nodes.shape[0]
    dbl = lambda shape, dt: [pltpu.VMEM(shape, dt), pltpu.VMEM(shape, dt)]
    fn = pl.kernel(
        _main_body,
        out_type=[
            jax.ShapeDtypeStruct((n, _NCOLS_S), jnp.float32),
            jax.ShapeDtypeStruct((3, n, _NCOLS_S), jnp.float32),
        ],
        mesh=plsc.VectorSubcoreMesh(**_MESH),
        scratch_types=(
            dbl((_BC,), jnp.int32)
            + dbl((_BC, _L), jnp.float32)
            + dbl((_BC, _NCOLS_S), jnp.float32)
            + dbl((_BC, _NCOLS_V), jnp.float32)
            + dbl((3, _BC, _NCOLS_S), jnp.float32)
            + [pltpu.VMEM_SHARED((n,), jnp.int32),
               pltpu.VMEM((n // 10,), jnp.int32)]
            + [pltpu.SemaphoreType.DMA] * 6
        ),
        compiler_params=pltpu.CompilerParams(use_tc_tiling_on_sc=False, needs_layout_passes=False),
    )
    return fn(nodes, r16, s_tab, v_tab)


def kernel(nodes, R, embed_s, embed_v):
    n = nodes.shape[0]
    nodes_i = nodes.astype(jnp.int32)
    r16 = jnp.pad(R.astype(jnp.float32).reshape(n, 9), ((0, 0), (0, 7)))
    s_tab, v_tab = _prep(embed_s, embed_v)
    s, rv_plan = _main(nodes_i, r16, s_tab, v_tab)
    return s, jnp.transpose(rv_plan, (1, 2, 0))
